# baseline (speedup 1.0000x reference)
import functools

import jax
import jax.numpy as jnp
from jax import lax
from jax.experimental import pallas as pl
from jax.experimental.pallas import tpu as pltpu
from jax.experimental.pallas import tpu_sc as plsc

B, C, H, W = 4, 64, 256, 256
HW = H * W
N = 100000
NC, NS, L = 2, 16, 16
NW = NC * NS
TPB = NW // B
CPT = C // TPB
CH = 2000
NCH = N // CH
VECS = CH // L
P1CH = 2 * N // CH
NSEG = 16
SEG = HW // NSEG
NOUT = 8
PF = plsc.PackFormat.INTERLEAVED


def _point_math(cxv, cyv):
    ix = cxv * 255.0
    iy = cyv * 255.0
    xi = jnp.minimum(ix.astype(jnp.int32), W - 2)
    yi = jnp.minimum(iy.astype(jnp.int32), H - 2)
    wx = ix - xi.astype(jnp.float32)
    wy = iy - yi.astype(jnp.float32)
    return yi * W + xi, wx, wy


def _sc_bilinear(feat1, cx, cy):
    mesh = plsc.VectorSubcoreMesh(core_axis_name="c", subcore_axis_name="s")

    @functools.partial(
        pl.kernel,
        out_type=jax.ShapeDtypeStruct((B * C * N,), jnp.float32),
        mesh=mesh,
        compiler_params=pltpu.CompilerParams(needs_layout_passes=False),
        scratch_types=(
            [pltpu.VMEM((HW,), jnp.int32)]
            + [pltpu.VMEM((CH,), jnp.int32) for _ in range(4)]
            + [pltpu.VMEM((CH,), jnp.int32) for _ in range(4)]
            + [pltpu.VMEM((CH,), jnp.float32) for _ in range(NOUT)]
            + [pltpu.VMEM_SHARED((2 * N,), jnp.int32),
               pltpu.VMEM_SHARED((2 * N,), jnp.int32)]
            + [pltpu.SemaphoreType.DMA for _ in range(8 + NOUT + 1)]
        ),
    )
    def k(feat_hbm, cx_hbm, cy_hbm, out_hbm, plane_v,
          idx0_v, idx1_v, idx2_v, idx3_v, w0_v, w1_v, w2_v, w3_v,
          o0, o1, o2, o3, o4, o5, o6, o7,
          sp_idx, sp_w,
          si0, si1, si2, si3, sw0, sw1, sw2, sw3,
          q0, q1, q2, q3, q4, q5, q6, q7, sp):
        scid = lax.axis_index("c")
        sid = lax.axis_index("s")
        sidx = (si0, si1, si2, si3)
        sw = (sw0, sw1, sw2, sw3)
        sout = (q0, q1, q2, q3, q4, q5, q6, q7)
        idxb_ = (idx0_v, idx1_v, idx2_v, idx3_v)
        wb_ = (w0_v, w1_v, w2_v, w3_v)
        outb_ = (o0, o1, o2, o3, o4, o5, o6, o7)

        nch_t = (P1CH - 1 - sid) // NS + 1

        def p1_body(j, carry):
            pool_off = (sid + NS * j) * CH
            gbase = scid * (2 * N) + pool_off
            pltpu.async_copy(cx_hbm.at[pl.ds(gbase, CH)], o0, q0)
            pltpu.async_copy(cy_hbm.at[pl.ds(gbase, CH)], o1, q1)
            pltpu.make_async_copy(cx_hbm.at[pl.ds(gbase, CH)], o0, q0).wait()
            pltpu.make_async_copy(cy_hbm.at[pl.ds(gbase, CH)], o1, q1).wait()

            @plsc.parallel_loop(0, VECS, unroll=5)
            def p1_vec(i):
                s = pl.ds(i * L, L)
                i00, wx, wy = _point_math(o0[s], o1[s])
                idx0_v[s] = i00
                w0_v[s] = plsc.bitcast(plsc.pack(wx, wy, format=PF),
                                       jnp.int32)

            pltpu.sync_copy(idx0_v, sp_idx.at[pl.ds(pool_off, CH)])
            pltpu.sync_copy(w0_v, sp_w.at[pl.ds(pool_off, CH)])
            return carry

        lax.fori_loop(0, nch_t, p1_body, 0)
        plsc.subcore_barrier()

        b_local = sid // TPB
        bb = scid * 2 + b_local
        cg = sid % TPB
        pool0 = b_local * N

        def issue_chunk(kk, bix):
            poff = pool0 + kk * CH
            pltpu.async_copy(sp_idx.at[pl.ds(poff, CH)], idxb_[bix],
                             sidx[bix])
            pltpu.async_copy(sp_w.at[pl.ds(poff, CH)], wb_[bix], sw[bix])

        def wait_chunk(kk, bix):
            poff = pool0 + kk * CH
            pltpu.make_async_copy(sp_idx.at[pl.ds(poff, CH)], idxb_[bix],
                                  sidx[bix]).wait()
            pltpu.make_async_copy(sp_w.at[pl.ds(poff, CH)], wb_[bix],
                                  sw[bix]).wait()

        def wait_out(plane_row, kk, oslot):
            obase = plane_row * N + kk * CH
            pltpu.make_async_copy(outb_[oslot],
                                  out_hbm.at[pl.ds(obase, CH)],
                                  sout[oslot]).wait()

        def chan_body(ci, carry):
            plane_row = bb * C + cg * CPT + ci
            issue_chunk(0, 0)
            issue_chunk(1, 1)
            issue_chunk(2, 2)
            fbase = plane_row * HW
            for seg in range(NSEG):
                pltpu.async_copy(feat_hbm.at[pl.ds(fbase + seg * SEG, SEG)],
                                 plane_v.at[pl.ds(seg * SEG, SEG)], sp)
            for seg in range(NSEG):
                pltpu.make_async_copy(
                    feat_hbm.at[pl.ds(fbase + seg * SEG, SEG)],
                    plane_v.at[pl.ds(seg * SEG, SEG)], sp).wait()

            def do_chunk(kk, bix, oslot, guard):
                if guard is True:
                    issue_chunk(kk + 3, (bix + 3) % 4)
                elif guard is not None:
                    @pl.when(guard)
                    def _pf():
                        issue_chunk(kk + 3, (bix + 3) % 4)
                wait_chunk(kk, bix)

                idxb = idxb_[bix]
                wb = wb_[bix]
                outb = outb_[oslot]

                @plsc.parallel_loop(0, VECS, unroll=5)
                def vec_body(i):
                    s = pl.ds(i * L, L)
                    i00 = idxb[s]
                    wx, wy = plsc.unpack(
                        plsc.bitcast(wb[s], jnp.bfloat16), format=PF)
                    p0 = plsc.load_gather(plane_v, [i00])
                    p1 = plsc.load_gather(plane_v, [i00 + W])
                    g00, g01 = plsc.unpack(
                        plsc.bitcast(p0, jnp.bfloat16), format=PF)
                    g10, g11 = plsc.unpack(
                        plsc.bitcast(p1, jnp.bfloat16), format=PF)
                    t0 = g00 + wx * (g01 - g00)
                    t1 = g10 + wx * (g11 - g10)
                    outb[s] = t0 + wy * (t1 - t0)

                obase = plane_row * N + kk * CH
                pltpu.async_copy(outb_[oslot],
                                 out_hbm.at[pl.ds(obase, CH)],
                                 sout[oslot])

            def chunk8_body(g, carry2):
                for slot in range(NOUT):
                    kk = g * NOUT + slot

                    @pl.when(g >= 1)
                    def _wait_out():
                        wait_out(plane_row, kk - NOUT, slot)

                    do_chunk(kk, slot % 4, slot,
                             guard=(kk + 3 < NCH) if slot == 7 else True)
                return carry2

            lax.fori_loop(0, NCH // NOUT, chunk8_body, 0)
            for slot in range(NCH % NOUT):
                kk = (NCH // NOUT) * NOUT + slot
                wait_out(plane_row, kk - NOUT, slot)
                do_chunk(kk, kk % 4, slot, guard=None)
            for kk in range(NCH - NOUT, NCH):
                wait_out(plane_row, kk, kk % NOUT)
            return carry

        lax.fori_loop(0, CPT, chan_body, 0)

    return k(feat1, cx, cy)


def kernel(grid_feat, grid_coord):
    fb = grid_feat.reshape(B * C, HW).astype(jnp.bfloat16)
    lo = jax.lax.bitcast_convert_type(fb, jnp.uint16).astype(jnp.uint32)
    fn = jnp.concatenate([fb[:, 1:], fb[:, -1:]], axis=1)
    hi = jax.lax.bitcast_convert_type(fn, jnp.uint16).astype(jnp.uint32)
    pairs = ((hi << 16) | lo).astype(jnp.int32).reshape(B * C * HW)
    cx = grid_coord[:, :, 0].reshape(B * N)
    cy = grid_coord[:, :, 1].reshape(B * N)
    out = _sc_bilinear(pairs, cx, cy)
    return out.reshape(B, C, N)


# speedup vs baseline: 1.4168x; 1.4168x over previous
import functools

import jax
import jax.numpy as jnp
from jax import lax
from jax.experimental import pallas as pl
from jax.experimental.pallas import tpu as pltpu
from jax.experimental.pallas import tpu_sc as plsc

B, C, H, W = 4, 64, 256, 256
HW = H * W
N = 100000
NC, NS, L = 2, 16, 16
NW = NC * NS
TPB = NW // B
CPT = C // TPB
CH = 2000
NCH = N // CH
VECS = CH // L
P1CH = 2 * N // CH
NSEG = 16
SEG = HW // NSEG
NOUT = 8
PF = plsc.PackFormat.INTERLEAVED


def _point_math(cxv, cyv):
    ix = cxv * 255.0
    iy = cyv * 255.0
    xi = jnp.minimum(ix.astype(jnp.int32), W - 2)
    yi = jnp.minimum(iy.astype(jnp.int32), H - 2)
    wx = ix - xi.astype(jnp.float32)
    wy = iy - yi.astype(jnp.float32)
    return yi * W + xi, wx, wy


def _sc_bilinear(feat1, cx, cy):
    mesh = plsc.VectorSubcoreMesh(core_axis_name="c", subcore_axis_name="s")

    @functools.partial(
        pl.kernel,
        out_type=jax.ShapeDtypeStruct((B * C * N,), jnp.float32),
        mesh=mesh,
        compiler_params=pltpu.CompilerParams(needs_layout_passes=False),
        scratch_types=(
            [pltpu.VMEM((HW,), jnp.float32)]
            + [pltpu.VMEM((CH,), jnp.int32) for _ in range(4)]
            + [pltpu.VMEM((CH,), jnp.int32) for _ in range(4)]
            + [pltpu.VMEM((CH,), jnp.float32) for _ in range(NOUT)]
            + [pltpu.VMEM_SHARED((2 * N,), jnp.int32),
               pltpu.VMEM_SHARED((2 * N,), jnp.int32)]
            + [pltpu.SemaphoreType.DMA for _ in range(8 + NOUT + 1)]
        ),
    )
    def k(feat_hbm, cx_hbm, cy_hbm, out_hbm, plane_v,
          idx0_v, idx1_v, idx2_v, idx3_v, w0_v, w1_v, w2_v, w3_v,
          o0, o1, o2, o3, o4, o5, o6, o7,
          sp_idx, sp_w,
          si0, si1, si2, si3, sw0, sw1, sw2, sw3,
          q0, q1, q2, q3, q4, q5, q6, q7, sp):
        scid = lax.axis_index("c")
        sid = lax.axis_index("s")
        sidx = (si0, si1, si2, si3)
        sw = (sw0, sw1, sw2, sw3)
        sout = (q0, q1, q2, q3, q4, q5, q6, q7)
        idxb_ = (idx0_v, idx1_v, idx2_v, idx3_v)
        wb_ = (w0_v, w1_v, w2_v, w3_v)
        outb_ = (o0, o1, o2, o3, o4, o5, o6, o7)

        nch_t = (P1CH - 1 - sid) // NS + 1

        def p1_body(j, carry):
            pool_off = (sid + NS * j) * CH
            gbase = scid * (2 * N) + pool_off
            pltpu.async_copy(cx_hbm.at[pl.ds(gbase, CH)], o0, q0)
            pltpu.async_copy(cy_hbm.at[pl.ds(gbase, CH)], o1, q1)
            pltpu.make_async_copy(cx_hbm.at[pl.ds(gbase, CH)], o0, q0).wait()
            pltpu.make_async_copy(cy_hbm.at[pl.ds(gbase, CH)], o1, q1).wait()

            @plsc.parallel_loop(0, VECS, unroll=5)
            def p1_vec(i):
                s = pl.ds(i * L, L)
                i00, wx, wy = _point_math(o0[s], o1[s])
                idx0_v[s] = i00
                w0_v[s] = plsc.bitcast(plsc.pack(wx, wy, format=PF),
                                       jnp.int32)

            pltpu.sync_copy(idx0_v, sp_idx.at[pl.ds(pool_off, CH)])
            pltpu.sync_copy(w0_v, sp_w.at[pl.ds(pool_off, CH)])
            return carry

        lax.fori_loop(0, nch_t, p1_body, 0)
        plsc.subcore_barrier()

        b_local = sid // TPB
        bb = scid * 2 + b_local
        cg = sid % TPB
        pool0 = b_local * N

        def issue_chunk(kk, bix):
            poff = pool0 + kk * CH
            pltpu.async_copy(sp_idx.at[pl.ds(poff, CH)], idxb_[bix],
                             sidx[bix])
            pltpu.async_copy(sp_w.at[pl.ds(poff, CH)], wb_[bix], sw[bix])

        def wait_chunk(kk, bix):
            poff = pool0 + kk * CH
            pltpu.make_async_copy(sp_idx.at[pl.ds(poff, CH)], idxb_[bix],
                                  sidx[bix]).wait()
            pltpu.make_async_copy(sp_w.at[pl.ds(poff, CH)], wb_[bix],
                                  sw[bix]).wait()

        def wait_out(plane_row, kk, oslot):
            obase = plane_row * N + kk * CH
            pltpu.make_async_copy(outb_[oslot],
                                  out_hbm.at[pl.ds(obase, CH)],
                                  sout[oslot]).wait()

        def chan_body(ci, carry):
            plane_row = bb * C + cg * CPT + ci
            issue_chunk(0, 0)
            issue_chunk(1, 1)
            issue_chunk(2, 2)
            fbase = plane_row * HW
            for seg in range(NSEG):
                pltpu.async_copy(feat_hbm.at[pl.ds(fbase + seg * SEG, SEG)],
                                 plane_v.at[pl.ds(seg * SEG, SEG)], sp)
            for seg in range(NSEG):
                pltpu.make_async_copy(
                    feat_hbm.at[pl.ds(fbase + seg * SEG, SEG)],
                    plane_v.at[pl.ds(seg * SEG, SEG)], sp).wait()

            def do_chunk(kk, bix, oslot, guard):
                if guard is True:
                    issue_chunk(kk + 3, (bix + 3) % 4)
                elif guard is not None:
                    @pl.when(guard)
                    def _pf():
                        issue_chunk(kk + 3, (bix + 3) % 4)
                wait_chunk(kk, bix)

                idxb = idxb_[bix]
                wb = wb_[bix]
                outb = outb_[oslot]

                @plsc.parallel_loop(0, VECS, unroll=5)
                def vec_body(i):
                    s = pl.ds(i * L, L)
                    i00 = idxb[s]
                    wx, wy = plsc.unpack(
                        plsc.bitcast(wb[s], jnp.bfloat16), format=PF)
                    g00 = plsc.load_gather(plane_v, [i00])
                    g01 = plsc.load_gather(plane_v, [i00 + 1])
                    g10 = plsc.load_gather(plane_v, [i00 + W])
                    g11 = plsc.load_gather(plane_v, [i00 + (W + 1)])
                    t0 = g00 + wx * (g01 - g00)
                    t1 = g10 + wx * (g11 - g10)
                    outb[s] = t0 + wy * (t1 - t0)

                obase = plane_row * N + kk * CH
                pltpu.async_copy(outb_[oslot],
                                 out_hbm.at[pl.ds(obase, CH)],
                                 sout[oslot])

            def chunk8_body(g, carry2):
                for slot in range(NOUT):
                    kk = g * NOUT + slot

                    @pl.when(g >= 1)
                    def _wait_out():
                        wait_out(plane_row, kk - NOUT, slot)

                    do_chunk(kk, slot % 4, slot,
                             guard=(kk + 3 < NCH) if slot == 7 else True)
                return carry2

            lax.fori_loop(0, NCH // NOUT, chunk8_body, 0)
            for slot in range(NCH % NOUT):
                kk = (NCH // NOUT) * NOUT + slot
                wait_out(plane_row, kk - NOUT, slot)
                do_chunk(kk, kk % 4, slot, guard=None)
            for kk in range(NCH - NOUT, NCH):
                wait_out(plane_row, kk, kk % NOUT)
            return carry

        lax.fori_loop(0, CPT, chan_body, 0)

    return k(feat1, cx, cy)


def kernel(grid_feat, grid_coord):
    feat1 = grid_feat.reshape(B * C * HW)
    cx = grid_coord[:, :, 0].reshape(B * N)
    cy = grid_coord[:, :, 1].reshape(B * N)
    out = _sc_bilinear(feat1, cx, cy)
    return out.reshape(B, C, N)
